# Initial kernel scaffold; baseline (speedup 1.0000x reference)
import jax
import jax.numpy as jnp
from jax.experimental import pallas as pl


def kernel(x, node_time, seed_time, batch_idx, edge_index, W_enc, b_enc, id_aware, w_time, b_time, W_self1, W_neigh1, b1, W_self2, W_neigh2, b2, lhs_W, lhs_b, rhs_emb):
    raise NotImplementedError("write your pallas kernel here")


# speedup vs baseline: 3.8605x; 3.8605x over previous
import functools

import jax
import jax.numpy as jnp
from jax import lax
from jax.experimental import pallas as pl
from jax.experimental.pallas import tpu as pltpu
from jax.experimental.pallas import tpu_sc as plsc

N = 10000
E = 320000
C = 128
D_FEAT = 128
EMB = 64
B = 256
NUM_RHS = 100000

NC = 2
NS = 16
NW = NC * NS
EPT = E // NW
K = 80
NPAD = 10240
RPT = NPAD // NS
ZB = 128
RT = 2048


def _sc_agg(h, src, dst):
    mesh = plsc.VectorSubcoreMesh(core_axis_name="c", subcore_axis_name="s")

    @functools.partial(
        pl.kernel,
        mesh=mesh,
        out_type=jax.ShapeDtypeStruct((NC * NPAD, C), jnp.float32),
        scratch_types=[
            pltpu.VMEM((K,), jnp.int32),
            pltpu.VMEM((K,), jnp.int32),
            pltpu.VMEM((K, C), jnp.float32),
            pltpu.VMEM((ZB, C), jnp.float32),
            pltpu.VMEM_SHARED((NPAD, C), jnp.float32),
            pltpu.SemaphoreType.DMA,
        ],
    )
    def agg(h_hbm, src_hbm, dst_hbm, out_hbm, src_v, dst_v, rows_v, zb_v,
            acc_sh, sem):
        c = lax.axis_index("c")
        s = lax.axis_index("s")
        row0 = s * RPT

        def zrow(r, carry):
            def zcol(j, carry2):
                zb_v[r, pl.ds(j * 16, 16)] = jnp.zeros((16,), jnp.float32)
                return carry2
            return lax.fori_loop(0, C // 16, zcol, carry)
        lax.fori_loop(0, ZB, zrow, 0)
        for t in range(RPT // ZB):
            pltpu.sync_copy(zb_v, acc_sh.at[pl.ds(row0 + t * ZB, ZB)])
        plsc.subcore_barrier()

        base = (c * NS + s) * EPT

        def body(i, carry):
            off = base + i * K
            pltpu.sync_copy(src_hbm.at[pl.ds(off, K)], src_v)
            pltpu.sync_copy(dst_hbm.at[pl.ds(off, K)], dst_v)
            pltpu.async_copy(h_hbm.at[src_v], rows_v, sem).wait()
            pltpu.sync_copy(rows_v, acc_sh.at[dst_v], add=True)
            return carry

        lax.fori_loop(0, EPT // K, body, 0)
        plsc.subcore_barrier()

        for t in range(RPT // ZB):
            pltpu.sync_copy(acc_sh.at[pl.ds(row0 + t * ZB, ZB)], zb_v)
            pltpu.sync_copy(
                zb_v, out_hbm.at[pl.ds(c * NPAD + row0 + t * ZB, ZB)])

    return agg(h, src, dst)


def _encoder(x, node_time, seed_time, batch_idx, W_enc, b_enc, id_aware,
             w_time, b_time):
    def body(x_ref, nt_ref, st_ref, bi_ref, W_ref, be_ref, ia_ref, wt_ref,
             bt_ref, o_ref):
        h = jnp.dot(x_ref[...], W_ref[...], preferred_element_type=jnp.float32)
        sel = bi_ref[...] == lax.broadcasted_iota(jnp.int32, (N, B), 1)
        st = jnp.sum(jnp.where(sel, st_ref[...], 0.0), axis=1, keepdims=True)
        rel = st - nt_ref[...]
        rowid = lax.broadcasted_iota(jnp.int32, (N, 1), 0)
        h = h + be_ref[...] + jnp.where(rowid < B, 1.0, 0.0) * ia_ref[...]
        o_ref[...] = h + rel * wt_ref[...] + bt_ref[...]

    return pl.pallas_call(
        body,
        out_shape=jax.ShapeDtypeStruct((N, C), jnp.float32),
    )(x, node_time.reshape(N, 1), seed_time.reshape(1, B),
      batch_idx.reshape(N, 1), W_enc, b_enc.reshape(1, C),
      id_aware.reshape(1, C), w_time.reshape(1, C), b_time.reshape(1, C))


def _sage1(h0, p, W_self1, W_neigh1, b1):
    def body(h_ref, p_ref, ws_ref, wn_ref, b_ref, o_ref):
        agg = p_ref[0:N, :] + p_ref[NPAD:NPAD + N, :]
        o_ref[...] = jax.nn.relu(
            jnp.dot(h_ref[...], ws_ref[...], preferred_element_type=jnp.float32)
            + jnp.dot(agg, wn_ref[...], preferred_element_type=jnp.float32)
            + b_ref[...])

    return pl.pallas_call(
        body,
        out_shape=jax.ShapeDtypeStruct((N, C), jnp.float32),
    )(h0, p, W_self1, W_neigh1, b1.reshape(1, C))


def _head(h1b, a0, a1, W_self2, W_neigh2, b2, lhs_W, lhs_b):
    def body(h_ref, a0_ref, a1_ref, ws_ref, wn_ref, b_ref, lw_ref, lb_ref,
             o_ref):
        agg = a0_ref[...] + a1_ref[...]
        h2 = (jnp.dot(h_ref[...], ws_ref[...],
                      preferred_element_type=jnp.float32)
              + jnp.dot(agg, wn_ref[...], preferred_element_type=jnp.float32)
              + b_ref[...])
        o_ref[...] = (jnp.dot(h2, lw_ref[...],
                              preferred_element_type=jnp.float32)
                      + lb_ref[...])

    return pl.pallas_call(
        body,
        out_shape=jax.ShapeDtypeStruct((B, EMB), jnp.float32),
    )(h1b, a0, a1, W_self2, W_neigh2, b2.reshape(1, C), lhs_W,
      lhs_b.reshape(1, EMB))


def _score(lhs, rhs_emb):
    def body(l_ref, r_ref, o_ref):
        o_ref[...] = lax.dot_general(
            l_ref[...], r_ref[...], (((1,), (1,)), ((), ())),
            preferred_element_type=jnp.float32)

    return pl.pallas_call(
        body,
        grid=(pl.cdiv(NUM_RHS, RT),),
        in_specs=[
            pl.BlockSpec((B, EMB), lambda i: (0, 0)),
            pl.BlockSpec((RT, EMB), lambda i: (i, 0)),
        ],
        out_specs=pl.BlockSpec((B, RT), lambda i: (0, i)),
        out_shape=jax.ShapeDtypeStruct((B, NUM_RHS), jnp.float32),
    )(lhs, rhs_emb)


def kernel(x, node_time, seed_time, batch_idx, edge_index, W_enc, b_enc,
           id_aware, w_time, b_time, W_self1, W_neigh1, b1, W_self2,
           W_neigh2, b2, lhs_W, lhs_b, rhs_emb):
    batch_idx = batch_idx.astype(jnp.int32)
    src = edge_index[0]
    dst = edge_index[1]

    h0 = _encoder(x, node_time, seed_time, batch_idx, W_enc, b_enc,
                  id_aware, w_time, b_time)
    p1 = _sc_agg(h0, src, dst)
    h1 = _sage1(h0, p1, W_self1, W_neigh1, b1)
    p2 = _sc_agg(h1, src, dst)
    lhs = _head(h1[:B], p2[:B], p2[NPAD:NPAD + B], W_self2, W_neigh2, b2,
                lhs_W, lhs_b)
    return _score(lhs, rhs_emb)


# speedup vs baseline: 6.9602x; 1.8029x over previous
import functools

import jax
import jax.numpy as jnp
from jax import lax
from jax.experimental import pallas as pl
from jax.experimental.pallas import tpu as pltpu
from jax.experimental.pallas import tpu_sc as plsc

N = 10000
E = 320000
C = 128
D_FEAT = 128
EMB = 64
B = 256
NUM_RHS = 100000

NC = 2
NS = 16
NW = NC * NS
EPT = E // NW
K = 80
NCH = EPT // K
NPAD = 10240
RPT = NPAD // NS
ZB = 128
CAP = EPT + K + 16
TRASH = EPT + K
A2 = 2 * B
RT = 2048

_SC_MESH = dict(core_axis_name="c", subcore_axis_name="s")


def _sc_agg_full(h, src3, dstf):
    @functools.partial(
        pl.kernel,
        mesh=plsc.VectorSubcoreMesh(**_SC_MESH),
        out_type=jax.ShapeDtypeStruct((NC * NPAD, C), jnp.float32),
        scratch_types=[
            pltpu.VMEM((NCH, K), jnp.int32),
            pltpu.VMEM((K,), jnp.int32),
            pltpu.VMEM((K,), jnp.int32),
            pltpu.VMEM((K, C), jnp.float32),
            pltpu.VMEM((K, C), jnp.float32),
            pltpu.VMEM_SHARED((NPAD, C), jnp.float32),
            pltpu.SemaphoreType.DMA,
            pltpu.SemaphoreType.DMA,
            pltpu.SemaphoreType.DMA,
            pltpu.SemaphoreType.DMA,
            pltpu.SemaphoreType.DMA,
        ],
    )
    def agg(h_hbm, src_hbm, dst_hbm, out_hbm, src_v, d0, d1, r0, r1,
            acc_sh, semi, sg0, sg1, sd0, sd1):
        c = lax.axis_index("c")
        s = lax.axis_index("s")
        wid = c * NS + s
        row0 = s * RPT
        ebase = wid * EPT

        pltpu.async_copy(src_hbm.at[wid], src_v, semi)

        def zrow(r, carry):
            def zcol(j, carry2):
                r0[r, pl.ds(j * 16, 16)] = jnp.zeros((16,), jnp.float32)
                return carry2
            return lax.fori_loop(0, C // 16, zcol, carry)
        lax.fori_loop(0, K, zrow, 0)
        for t in range(RPT // K):
            pltpu.sync_copy(r0, acc_sh.at[pl.ds(row0 + t * K, K)])

        pltpu.make_async_copy(src_hbm.at[wid], src_v, semi).wait()

        pltpu.async_copy(h_hbm.at[src_v.at[0]], r0, sg0)
        pltpu.async_copy(h_hbm.at[src_v.at[1]], r1, sg1)
        pltpu.async_copy(dst_hbm.at[pl.ds(ebase, K)], d0, sd0)
        pltpu.async_copy(dst_hbm.at[pl.ds(ebase + K, K)], d1, sd1)
        plsc.subcore_barrier()

        def body(io, carry):
            ring = ((r0, sg0, d0, sd0), (r1, sg1, d1, sd1))
            for b, (rb, sgb, db, sdb) in enumerate(ring):
                ch = 2 * io + b
                pltpu.make_async_copy(h_hbm.at[src_v.at[ch]], rb, sgb).wait()
                pltpu.make_async_copy(
                    dst_hbm.at[pl.ds(ebase + ch * K, K)], db, sdb).wait()
                pltpu.sync_copy(rb, acc_sh.at[db], add=True)
                nxt = ch + 2

                @pl.when(nxt < NCH)
                def _():
                    pltpu.async_copy(h_hbm.at[src_v.at[nxt]], rb, sgb)
                    pltpu.async_copy(
                        dst_hbm.at[pl.ds(ebase + nxt * K, K)], db, sdb)
            return carry

        lax.fori_loop(0, NCH // 2, body, 0)
        pltpu.make_async_copy(h_hbm.at[src_v.at[NCH - 1]], r0, sg0).wait()
        pltpu.make_async_copy(
            dst_hbm.at[pl.ds(ebase + (NCH - 1) * K, K)], d0, sd0).wait()
        pltpu.sync_copy(r0, acc_sh.at[d0], add=True)
        plsc.subcore_barrier()

        for t in range(RPT // K):
            pltpu.sync_copy(acc_sh.at[pl.ds(row0 + t * K, K)], r0)
            pltpu.sync_copy(
                r0, out_hbm.at[pl.ds(c * NPAD + row0 + t * K, K)])

    return agg(h, src3, dstf)


def _sc_agg_seed(h, src3, dst3):
    @functools.partial(
        pl.kernel,
        mesh=plsc.VectorSubcoreMesh(**_SC_MESH),
        out_type=jax.ShapeDtypeStruct((NC * B, C), jnp.float32),
        scratch_types=[
            pltpu.VMEM((NCH, K), jnp.int32),
            pltpu.VMEM((NCH, K), jnp.int32),
            pltpu.VMEM((CAP,), jnp.int32),
            pltpu.VMEM((CAP,), jnp.int32),
            pltpu.VMEM((K,), jnp.int32),
            pltpu.VMEM((K, C), jnp.float32),
            pltpu.VMEM((A2 // NS, C), jnp.float32),
            pltpu.VMEM_SHARED((A2, C), jnp.float32),
            pltpu.SemaphoreType.DMA,
            pltpu.SemaphoreType.DMA,
        ],
    )
    def agg(h_hbm, src_hbm, dst_hbm, out_hbm, src_v, dst_v, csrc, cdst,
            dstc_v, rows_v, zb_v, acc_sh, semi, sg):
        c = lax.axis_index("c")
        s = lax.axis_index("s")
        wid = c * NS + s

        pltpu.async_copy(src_hbm.at[wid], src_v, semi)
        pltpu.async_copy(dst_hbm.at[wid], dst_v, semi)

        def zrow(r, carry):
            def zcol(j, carry2):
                zb_v[r, pl.ds(j * 16, 16)] = jnp.zeros((16,), jnp.float32)
                return carry2
            return lax.fori_loop(0, C // 16, zcol, carry)
        lax.fori_loop(0, A2 // NS, zrow, 0)
        pltpu.sync_copy(zb_v, acc_sh.at[pl.ds(s * (A2 // NS), A2 // NS)])

        pltpu.make_async_copy(src_hbm.at[wid], src_v, semi).wait()
        pltpu.make_async_copy(dst_hbm.at[wid], dst_v, semi).wait()

        lane = lax.iota(jnp.int32, 16)

        def crow(r, offv):
            for j in range(K // 16):
                s16 = src_v[r, pl.ds(j * 16, 16)]
                d16 = dst_v[r, pl.ds(j * 16, 16)]
                m = d16 < B
                key = jnp.where(m, lane, 16 + lane)
                _, ssrc = plsc.sort_key_val(key, s16)
                _, sdst = plsc.sort_key_val(key, d16)
                idx = offv + lane
                plsc.store_scatter(csrc, [idx], ssrc)
                plsc.store_scatter(cdst, [idx], sdst)
                offv = offv + plsc.all_reduce_population_count(m)
            return offv

        cntv = lax.fori_loop(0, NCH, crow, jnp.zeros((16,), jnp.int32))

        for j in range(K // 16):
            plsc.store_scatter(csrc, [cntv + (j * 16 + lane)],
                               jnp.zeros((16,), jnp.int32))
            plsc.store_scatter(cdst, [cntv + (j * 16 + lane)],
                               B + ((lane + j * 16) & (B - 1)))

        plsc.subcore_barrier()

        def body(ch, carry):
            @pl.when(jnp.any(cntv > ch * K))
            def _():
                for j in range(K // 16):
                    dstc_v[pl.ds(j * 16, 16)] = (
                        cdst[pl.ds(ch * K + j * 16, 16)])
                pltpu.async_copy(
                    h_hbm.at[csrc.at[pl.ds(ch * K, K)]], rows_v, sg).wait()
                pltpu.sync_copy(rows_v, acc_sh.at[dstc_v], add=True)
            return carry

        lax.fori_loop(0, NCH, body, 0)
        plsc.subcore_barrier()

        pltpu.sync_copy(acc_sh.at[pl.ds(s * 16, 16)], zb_v.at[pl.ds(0, 16)])
        pltpu.sync_copy(zb_v.at[pl.ds(0, 16)],
                        out_hbm.at[pl.ds(c * B + s * 16, 16)])

    return agg(h, src3, dst3)


def _encoder(x, node_time, seed_time, batch_idx, W_enc, b_enc, id_aware,
             w_time, b_time):
    def body(x_ref, nt_ref, st_ref, bi_ref, W_ref, be_ref, ia_ref, wt_ref,
             bt_ref, o_ref):
        h = jnp.dot(x_ref[...], W_ref[...], preferred_element_type=jnp.float32)
        sel = bi_ref[...] == lax.broadcasted_iota(jnp.int32, (N, B), 1)
        st = jnp.sum(jnp.where(sel, st_ref[...], 0.0), axis=1, keepdims=True)
        rel = st - nt_ref[...]
        rowid = lax.broadcasted_iota(jnp.int32, (N, 1), 0)
        h = h + be_ref[...] + jnp.where(rowid < B, 1.0, 0.0) * ia_ref[...]
        o_ref[...] = h + rel * wt_ref[...] + bt_ref[...]

    return pl.pallas_call(
        body,
        out_shape=jax.ShapeDtypeStruct((N, C), jnp.float32),
    )(x, node_time.reshape(N, 1), seed_time.reshape(1, B),
      batch_idx.reshape(N, 1), W_enc, b_enc.reshape(1, C),
      id_aware.reshape(1, C), w_time.reshape(1, C), b_time.reshape(1, C))


def _sage1(h0, p, W_self1, W_neigh1, b1):
    def body(h_ref, p_ref, ws_ref, wn_ref, b_ref, o_ref):
        agg = p_ref[0:N, :] + p_ref[NPAD:NPAD + N, :]
        o_ref[...] = jax.nn.relu(
            jnp.dot(h_ref[...], ws_ref[...], preferred_element_type=jnp.float32)
            + jnp.dot(agg, wn_ref[...], preferred_element_type=jnp.float32)
            + b_ref[...])

    return pl.pallas_call(
        body,
        out_shape=jax.ShapeDtypeStruct((N, C), jnp.float32),
    )(h0, p, W_self1, W_neigh1, b1.reshape(1, C))


def _head(h1b, a0, a1, W_self2, W_neigh2, b2, lhs_W, lhs_b):
    def body(h_ref, a0_ref, a1_ref, ws_ref, wn_ref, b_ref, lw_ref, lb_ref,
             o_ref):
        agg = a0_ref[...] + a1_ref[...]
        h2 = (jnp.dot(h_ref[...], ws_ref[...],
                      preferred_element_type=jnp.float32)
              + jnp.dot(agg, wn_ref[...], preferred_element_type=jnp.float32)
              + b_ref[...])
        o_ref[...] = (jnp.dot(h2, lw_ref[...],
                              preferred_element_type=jnp.float32)
                      + lb_ref[...])

    return pl.pallas_call(
        body,
        out_shape=jax.ShapeDtypeStruct((B, EMB), jnp.float32),
    )(h1b, a0, a1, W_self2, W_neigh2, b2.reshape(1, C), lhs_W,
      lhs_b.reshape(1, EMB))


def _score(lhs, rhs_emb):
    def body(l_ref, r_ref, o_ref):
        o_ref[...] = lax.dot_general(
            l_ref[...], r_ref[...], (((1,), (1,)), ((), ())),
            preferred_element_type=jnp.float32)

    return pl.pallas_call(
        body,
        grid=(pl.cdiv(NUM_RHS, RT),),
        in_specs=[
            pl.BlockSpec((B, EMB), lambda i: (0, 0)),
            pl.BlockSpec((RT, EMB), lambda i: (i, 0)),
        ],
        out_specs=pl.BlockSpec((B, RT), lambda i: (0, i)),
        out_shape=jax.ShapeDtypeStruct((B, NUM_RHS), jnp.float32),
    )(lhs, rhs_emb)


def kernel(x, node_time, seed_time, batch_idx, edge_index, W_enc, b_enc,
           id_aware, w_time, b_time, W_self1, W_neigh1, b1, W_self2,
           W_neigh2, b2, lhs_W, lhs_b, rhs_emb):
    batch_idx = batch_idx.astype(jnp.int32)
    src3 = edge_index[0].reshape(NW, NCH, K)
    dst3 = edge_index[1].reshape(NW, NCH, K)

    h0 = _encoder(x, node_time, seed_time, batch_idx, W_enc, b_enc,
                  id_aware, w_time, b_time)
    p1 = _sc_agg_full(h0, src3, edge_index[1])
    h1 = _sage1(h0, p1, W_self1, W_neigh1, b1)
    p2 = _sc_agg_full(h1, src3, edge_index[1])
    lhs = _head(h1[:B], p2[:B], p2[NPAD:NPAD + B], W_self2, W_neigh2, b2,
                lhs_W, lhs_b)
    return _score(lhs, rhs_emb)


# speedup vs baseline: 7.1402x; 1.0259x over previous
import functools

import jax
import jax.numpy as jnp
from jax import lax
from jax.experimental import pallas as pl
from jax.experimental.pallas import tpu as pltpu
from jax.experimental.pallas import tpu_sc as plsc

N = 10000
E = 320000
C = 128
D_FEAT = 128
EMB = 64
B = 256
NUM_RHS = 100000

NC = 2
NS = 16
NW = NC * NS
EPT = E // NW
K = 80
NCH = EPT // K
NPAD = 10240
RPT = NPAD // NS
ZB = 128
CAP = EPT + K + 16
TRASH = EPT + K
A2 = 2 * B
RT = 2048

_SC_MESH = dict(core_axis_name="c", subcore_axis_name="s")


def _sc_agg_full(h, src3, dstf):
    @functools.partial(
        pl.kernel,
        mesh=plsc.VectorSubcoreMesh(**_SC_MESH),
        out_type=jax.ShapeDtypeStruct((NC * NPAD, C), jnp.float32),
        scratch_types=[
            pltpu.VMEM((NCH, K), jnp.int32),
            pltpu.VMEM((K,), jnp.int32),
            pltpu.VMEM((K,), jnp.int32),
            pltpu.VMEM((K, C), jnp.float32),
            pltpu.VMEM((K, C), jnp.float32),
            pltpu.VMEM_SHARED((NPAD, C), jnp.float32),
            pltpu.SemaphoreType.DMA,
            pltpu.SemaphoreType.DMA,
            pltpu.SemaphoreType.DMA,
            pltpu.SemaphoreType.DMA,
            pltpu.SemaphoreType.DMA,
        ],
    )
    def agg(h_hbm, src_hbm, dst_hbm, out_hbm, src_v, d0, d1, r0, r1,
            acc_sh, semi, sg0, sg1, sd0, sd1):
        c = lax.axis_index("c")
        s = lax.axis_index("s")
        wid = c * NS + s
        row0 = s * RPT
        ebase = wid * EPT

        pltpu.async_copy(src_hbm.at[wid], src_v, semi)

        def zrow(r, carry):
            def zcol(j, carry2):
                r0[r, pl.ds(j * 16, 16)] = jnp.zeros((16,), jnp.float32)
                return carry2
            return lax.fori_loop(0, C // 16, zcol, carry)
        lax.fori_loop(0, K, zrow, 0)
        for t in range(RPT // K):
            pltpu.sync_copy(r0, acc_sh.at[pl.ds(row0 + t * K, K)])

        pltpu.make_async_copy(src_hbm.at[wid], src_v, semi).wait()

        pltpu.async_copy(h_hbm.at[src_v.at[0]], r0, sg0)
        pltpu.async_copy(h_hbm.at[src_v.at[1]], r1, sg1)
        pltpu.async_copy(dst_hbm.at[pl.ds(ebase, K)], d0, sd0)
        pltpu.async_copy(dst_hbm.at[pl.ds(ebase + K, K)], d1, sd1)
        plsc.subcore_barrier()

        def body(io, carry):
            ring = ((r0, sg0, d0, sd0), (r1, sg1, d1, sd1))
            for b, (rb, sgb, db, sdb) in enumerate(ring):
                ch = 2 * io + b
                pltpu.make_async_copy(h_hbm.at[src_v.at[ch]], rb, sgb).wait()
                pltpu.make_async_copy(
                    dst_hbm.at[pl.ds(ebase + ch * K, K)], db, sdb).wait()
                pltpu.sync_copy(rb, acc_sh.at[db], add=True)
                nxt = ch + 2

                @pl.when(nxt < NCH)
                def _():
                    pltpu.async_copy(h_hbm.at[src_v.at[nxt]], rb, sgb)
                    pltpu.async_copy(
                        dst_hbm.at[pl.ds(ebase + nxt * K, K)], db, sdb)
            return carry

        lax.fori_loop(0, NCH // 2, body, 0)
        pltpu.make_async_copy(h_hbm.at[src_v.at[NCH - 1]], r0, sg0).wait()
        pltpu.make_async_copy(
            dst_hbm.at[pl.ds(ebase + (NCH - 1) * K, K)], d0, sd0).wait()
        pltpu.sync_copy(r0, acc_sh.at[d0], add=True)
        plsc.subcore_barrier()

        for t in range(RPT // K):
            pltpu.sync_copy(acc_sh.at[pl.ds(row0 + t * K, K)], r0)
            pltpu.sync_copy(
                r0, out_hbm.at[pl.ds(c * NPAD + row0 + t * K, K)])

    return agg(h, src3, dstf)


def _sc_agg_seed(h, src3, dst3):
    @functools.partial(
        pl.kernel,
        mesh=plsc.VectorSubcoreMesh(**_SC_MESH),
        out_type=jax.ShapeDtypeStruct((NC * B, C), jnp.float32),
        scratch_types=[
            pltpu.VMEM((NCH, K), jnp.int32),
            pltpu.VMEM((NCH, K), jnp.int32),
            pltpu.VMEM((CAP,), jnp.int32),
            pltpu.VMEM((CAP,), jnp.int32),
            pltpu.VMEM((K,), jnp.int32),
            pltpu.VMEM((K, C), jnp.float32),
            pltpu.VMEM((A2 // NS, C), jnp.float32),
            pltpu.VMEM_SHARED((A2, C), jnp.float32),
            pltpu.SemaphoreType.DMA,
            pltpu.SemaphoreType.DMA,
        ],
    )
    def agg(h_hbm, src_hbm, dst_hbm, out_hbm, src_v, dst_v, csrc, cdst,
            dstc_v, rows_v, zb_v, acc_sh, semi, sg):
        c = lax.axis_index("c")
        s = lax.axis_index("s")
        wid = c * NS + s

        pltpu.async_copy(src_hbm.at[wid], src_v, semi)
        pltpu.async_copy(dst_hbm.at[wid], dst_v, semi)

        def zrow(r, carry):
            def zcol(j, carry2):
                zb_v[r, pl.ds(j * 16, 16)] = jnp.zeros((16,), jnp.float32)
                return carry2
            return lax.fori_loop(0, C // 16, zcol, carry)
        lax.fori_loop(0, A2 // NS, zrow, 0)
        pltpu.sync_copy(zb_v, acc_sh.at[pl.ds(s * (A2 // NS), A2 // NS)])

        pltpu.make_async_copy(src_hbm.at[wid], src_v, semi).wait()
        pltpu.make_async_copy(dst_hbm.at[wid], dst_v, semi).wait()

        lane = lax.iota(jnp.int32, 16)

        def crow(r, off):
            for j in range(K // 16):
                s16 = src_v[r, pl.ds(j * 16, 16)]
                d16 = dst_v[r, pl.ds(j * 16, 16)]
                m = d16 < B
                pad_rows = B + ((lane + (r * (K // 16) + j)) & (B - 1))
                csrc[pl.ds(off, 16)] = s16
                cdst[pl.ds(off, 16)] = jnp.where(m, d16, pad_rows)
                keep = jnp.where(m, jnp.int32(1), jnp.int32(0))
                t = keep + lax.rev(keep, (0,))
                cnt = (t[0] + t[1] + t[2] + t[3]
                       + t[4] + t[5] + t[6] + t[7])
                off = off + jnp.where(cnt > 0, 16, 0)
            return off

        cnt = lax.fori_loop(0, NCH, crow, jnp.int32(0))

        for j in range(K // 16 - 1):
            csrc[pl.ds(cnt + j * 16, 16)] = jnp.zeros((16,), jnp.int32)
            cdst[pl.ds(cnt + j * 16, 16)] = B + ((lane + j * 16) & (B - 1))

        plsc.subcore_barrier()

        def body(ch, carry):
            @pl.when(ch * K < cnt)
            def _():
                for j in range(K // 16):
                    dstc_v[pl.ds(j * 16, 16)] = (
                        cdst[pl.ds(ch * K + j * 16, 16)])
                pltpu.async_copy(
                    h_hbm.at[csrc.at[pl.ds(ch * K, K)]], rows_v, sg).wait()
                pltpu.sync_copy(rows_v, acc_sh.at[dstc_v], add=True)
            return carry

        lax.fori_loop(0, NCH, body, 0)
        plsc.subcore_barrier()

        pltpu.sync_copy(acc_sh.at[pl.ds(s * 16, 16)], zb_v.at[pl.ds(0, 16)])
        pltpu.sync_copy(zb_v.at[pl.ds(0, 16)],
                        out_hbm.at[pl.ds(c * B + s * 16, 16)])

    return agg(h, src3, dst3)


def _encoder(x, node_time, seed_time, batch_idx, W_enc, b_enc, id_aware,
             w_time, b_time):
    def body(x_ref, nt_ref, st_ref, bi_ref, W_ref, be_ref, ia_ref, wt_ref,
             bt_ref, o_ref):
        h = jnp.dot(x_ref[...], W_ref[...], preferred_element_type=jnp.float32)
        sel = bi_ref[...] == lax.broadcasted_iota(jnp.int32, (N, B), 1)
        st = jnp.sum(jnp.where(sel, st_ref[...], 0.0), axis=1, keepdims=True)
        rel = st - nt_ref[...]
        rowid = lax.broadcasted_iota(jnp.int32, (N, 1), 0)
        h = h + be_ref[...] + jnp.where(rowid < B, 1.0, 0.0) * ia_ref[...]
        o_ref[...] = h + rel * wt_ref[...] + bt_ref[...]

    return pl.pallas_call(
        body,
        out_shape=jax.ShapeDtypeStruct((N, C), jnp.float32),
    )(x, node_time.reshape(N, 1), seed_time.reshape(1, B),
      batch_idx.reshape(N, 1), W_enc, b_enc.reshape(1, C),
      id_aware.reshape(1, C), w_time.reshape(1, C), b_time.reshape(1, C))


def _sage1(h0, p, W_self1, W_neigh1, b1):
    def body(h_ref, p_ref, ws_ref, wn_ref, b_ref, o_ref):
        agg = p_ref[0:N, :] + p_ref[NPAD:NPAD + N, :]
        o_ref[...] = jax.nn.relu(
            jnp.dot(h_ref[...], ws_ref[...], preferred_element_type=jnp.float32)
            + jnp.dot(agg, wn_ref[...], preferred_element_type=jnp.float32)
            + b_ref[...])

    return pl.pallas_call(
        body,
        out_shape=jax.ShapeDtypeStruct((N, C), jnp.float32),
    )(h0, p, W_self1, W_neigh1, b1.reshape(1, C))


def _head(h1b, a0, a1, W_self2, W_neigh2, b2, lhs_W, lhs_b):
    def body(h_ref, a0_ref, a1_ref, ws_ref, wn_ref, b_ref, lw_ref, lb_ref,
             o_ref):
        agg = a0_ref[...] + a1_ref[...]
        h2 = (jnp.dot(h_ref[...], ws_ref[...],
                      preferred_element_type=jnp.float32)
              + jnp.dot(agg, wn_ref[...], preferred_element_type=jnp.float32)
              + b_ref[...])
        o_ref[...] = (jnp.dot(h2, lw_ref[...],
                              preferred_element_type=jnp.float32)
                      + lb_ref[...])

    return pl.pallas_call(
        body,
        out_shape=jax.ShapeDtypeStruct((B, EMB), jnp.float32),
    )(h1b, a0, a1, W_self2, W_neigh2, b2.reshape(1, C), lhs_W,
      lhs_b.reshape(1, EMB))


def _score(lhs, rhs_emb):
    def body(l_ref, r_ref, o_ref):
        o_ref[...] = lax.dot_general(
            l_ref[...], r_ref[...], (((1,), (1,)), ((), ())),
            preferred_element_type=jnp.float32)

    return pl.pallas_call(
        body,
        grid=(pl.cdiv(NUM_RHS, RT),),
        in_specs=[
            pl.BlockSpec((B, EMB), lambda i: (0, 0)),
            pl.BlockSpec((RT, EMB), lambda i: (i, 0)),
        ],
        out_specs=pl.BlockSpec((B, RT), lambda i: (0, i)),
        out_shape=jax.ShapeDtypeStruct((B, NUM_RHS), jnp.float32),
    )(lhs, rhs_emb)


def kernel(x, node_time, seed_time, batch_idx, edge_index, W_enc, b_enc,
           id_aware, w_time, b_time, W_self1, W_neigh1, b1, W_self2,
           W_neigh2, b2, lhs_W, lhs_b, rhs_emb):
    batch_idx = batch_idx.astype(jnp.int32)
    src3 = edge_index[0].reshape(NW, NCH, K)
    dst3 = edge_index[1].reshape(NW, NCH, K)

    h0 = _encoder(x, node_time, seed_time, batch_idx, W_enc, b_enc,
                  id_aware, w_time, b_time)
    p1 = _sc_agg_full(h0, src3, edge_index[1])
    h1 = _sage1(h0, p1, W_self1, W_neigh1, b1)
    p2 = _sc_agg_seed(h1, src3, dst3)
    lhs = _head(h1[:B], p2[:B], p2[B:2 * B], W_self2, W_neigh2, b2,
                lhs_W, lhs_b)
    return _score(lhs, rhs_emb)
